# baseline (device time: 41870 ns/iter reference)
import jax
import jax.numpy as jnp
from jax import lax
from jax.experimental import pallas as pl
from jax.experimental.pallas import tpu as pltpu

N_DEV = 4
SCALE = 0.08838834764831843
GQA_REP = 4


def kernel(x, Wq, Wo, K_ext, V_ext):
    B, Sq, D = x.shape
    Dq = Wq.shape[1]
    Dh = K_ext.shape[-1]
    Skv = K_ext.shape[1]
    n_local_heads = Dq // Dh
    n_local_kv = n_local_heads // GQA_REP
    CH = Sq // N_DEV

    xs = x[0]
    my_i = lax.axis_index("i")
    kv_heads = [
        lax.dynamic_index_in_dim(t, n_local_kv * my_i + g, axis=1,
                                 keepdims=False).astype(jnp.bfloat16)
        for t in (K_ext[0], V_ext[0])
        for g in range(n_local_kv)
    ]

    def body(x_ref, wq_ref, wo_ref, k0_ref, k1_ref, v0_ref, v1_ref, out_ref,
             wqb_ref, wob_ref,
             sbuf_rs, stage_rs, sbuf_ag, stage_ag,
             rs_send, rs_recv, ag_send, ag_recv, dummy_sem):
        kv_refs = ((k0_ref, v0_ref), (k1_ref, v1_ref))
        p = lax.axis_index("i")

        def mod4(v):
            return lax.rem(v + 2 * N_DEV, N_DEV)

        barrier_sem = pltpu.get_barrier_semaphore()
        for delta in (1, 2, 3):
            pl.semaphore_signal(barrier_sem, inc=1, device_id=(mod4(p + delta),),
                                device_id_type=pl.DeviceIdType.MESH)
        pl.semaphore_wait(barrier_sem, 3)

        wqb_ref[:, :] = wq_ref[:, :].astype(jnp.bfloat16)
        wob_ref[:, :] = wo_ref[:, :].astype(jnp.bfloat16)

        def compute_chunk(row):
            xb = x_ref[pl.ds(row, CH), :].astype(jnp.bfloat16)
            q_c = jnp.dot(xb, wqb_ref[:, :],
                          preferred_element_type=jnp.float32)
            q_c = (q_c * SCALE).astype(jnp.bfloat16)
            outs = []
            for h in range(n_local_heads):
                g = h // GQA_REP
                kg, vg = kv_refs[g]
                s = lax.dot_general(
                    q_c[:, h * Dh:(h + 1) * Dh], kg[:, :],
                    (((1,), (1,)), ((), ())),
                    preferred_element_type=jnp.float32)
                pj = jnp.exp(s)
                l = jnp.sum(pj, axis=1, keepdims=True)
                o = jnp.dot(pj.astype(jnp.bfloat16), vg[:, :],
                            preferred_element_type=jnp.float32) / l
                outs.append(o)
            attn = jnp.concatenate(outs, axis=1).astype(jnp.bfloat16)
            return jnp.dot(attn, wob_ref[:, :],
                           preferred_element_type=jnp.float32)

        for delta in (2, 1, 3):
            o = mod4(p + delta)
            part = compute_chunk(CH * o)
            myslot, dstslot = delta - 1, 3 - delta
            sbuf_rs[myslot] = part.astype(jnp.bfloat16)
            rdma = pltpu.make_async_remote_copy(
                src_ref=sbuf_rs.at[myslot],
                dst_ref=stage_rs.at[dstslot],
                send_sem=rs_send.at[myslot], recv_sem=rs_recv.at[dstslot],
                device_id=(o,), device_id_type=pl.DeviceIdType.MESH,
            )
            rdma.start()

        own = compute_chunk(CH * p)

        for j in range(3):
            rr = pltpu.make_async_remote_copy(
                src_ref=stage_rs.at[j], dst_ref=stage_rs.at[j],
                send_sem=dummy_sem.at[j], recv_sem=rs_recv.at[j],
                device_id=(p,), device_id_type=pl.DeviceIdType.MESH,
            )
            rr.wait_recv()

        red = (own + stage_rs[0].astype(jnp.float32)
               + stage_rs[1].astype(jnp.float32)
               + stage_rs[2].astype(jnp.float32))
        out_ref[pl.ds(CH * p, CH), :] = red
        sbuf_ag[:, :] = red.astype(jnp.bfloat16)

        for delta in (1, 2, 3):
            rdma = pltpu.make_async_remote_copy(
                src_ref=sbuf_ag,
                dst_ref=stage_ag.at[3 - delta],
                send_sem=ag_send.at[delta - 1], recv_sem=ag_recv.at[3 - delta],
                device_id=(mod4(p + delta),),
                device_id_type=pl.DeviceIdType.MESH,
            )
            rdma.start()

        for j in range(3):
            ra = pltpu.make_async_remote_copy(
                src_ref=stage_ag.at[j], dst_ref=stage_ag.at[j],
                send_sem=dummy_sem.at[j], recv_sem=ag_recv.at[j],
                device_id=(p,), device_id_type=pl.DeviceIdType.MESH,
            )
            ra.wait_recv()
            src_owner = mod4(p + j + 1)
            out_ref[pl.ds(CH * src_owner, CH), :] = stage_ag[j].astype(jnp.float32)

        for j in range(3):
            ws = pltpu.make_async_remote_copy(
                src_ref=sbuf_rs.at[j], dst_ref=stage_rs.at[j],
                send_sem=rs_send.at[j], recv_sem=dummy_sem.at[j],
                device_id=(p,), device_id_type=pl.DeviceIdType.MESH,
            )
            ws.wait_send()
            wa = pltpu.make_async_remote_copy(
                src_ref=sbuf_ag, dst_ref=stage_ag.at[j],
                send_sem=ag_send.at[j], recv_sem=dummy_sem.at[j],
                device_id=(p,), device_id_type=pl.DeviceIdType.MESH,
            )
            wa.wait_send()

    out = pl.pallas_call(
        body,
        out_shape=jax.ShapeDtypeStruct((Sq, D), jnp.float32),
        in_specs=[pl.BlockSpec(memory_space=pltpu.VMEM)] * 7,
        out_specs=pl.BlockSpec(memory_space=pltpu.VMEM),
        scratch_shapes=[
            pltpu.VMEM((Wq.shape[0], Dq), jnp.bfloat16),
            pltpu.VMEM((Dq, D), jnp.bfloat16),
            pltpu.VMEM((3, CH, D), jnp.bfloat16),
            pltpu.VMEM((3, CH, D), jnp.bfloat16),
            pltpu.VMEM((CH, D), jnp.bfloat16),
            pltpu.VMEM((3, CH, D), jnp.bfloat16),
            pltpu.SemaphoreType.DMA((3,)),
            pltpu.SemaphoreType.DMA((3,)),
            pltpu.SemaphoreType.DMA((3,)),
            pltpu.SemaphoreType.DMA((3,)),
            pltpu.SemaphoreType.DMA((3,)),
        ],
        compiler_params=pltpu.CompilerParams(collective_id=0),
    )(xs, Wq, Wo, *kv_heads)
    return out.reshape(B, Sq, D)


# device time: 36553 ns/iter; 1.1455x vs baseline; 1.1455x over previous
import jax
import jax.numpy as jnp
from jax import lax
from jax.experimental import pallas as pl
from jax.experimental.pallas import tpu as pltpu

N_DEV = 4
SCALE = 0.08838834764831843
GQA_REP = 4


def kernel(x, Wq, Wo, K_ext, V_ext):
    B, Sq, D = x.shape
    Dq = Wq.shape[1]
    Dh = K_ext.shape[-1]
    Skv = K_ext.shape[1]
    n_local_heads = Dq // Dh
    n_local_kv = n_local_heads // GQA_REP
    CH = Sq // N_DEV

    xs = x[0]
    my_i = lax.axis_index("i")
    Ks = lax.dynamic_slice_in_dim(K_ext[0], n_local_kv * my_i, n_local_kv,
                                  axis=1).astype(jnp.bfloat16).reshape(Skv, n_local_kv * Dh)
    Vs = lax.dynamic_slice_in_dim(V_ext[0], n_local_kv * my_i, n_local_kv,
                                  axis=1).astype(jnp.bfloat16).reshape(Skv, n_local_kv * Dh)

    def body(x_ref, wq_ref, wo_ref, kb_ref, vb_ref, out_ref,
             wqb_ref, wob_ref,
             sbuf_rs, stage_rs, sbuf_ag, stage_ag,
             rs_send, rs_recv, ag_send, ag_recv, dummy_sem):
        p = lax.axis_index("i")

        def mod4(v):
            return lax.rem(v + 2 * N_DEV, N_DEV)

        barrier_sem = pltpu.get_barrier_semaphore()
        for delta in (1, 2, 3):
            pl.semaphore_signal(barrier_sem, inc=1, device_id=(mod4(p + delta),),
                                device_id_type=pl.DeviceIdType.MESH)
        pl.semaphore_wait(barrier_sem, 3)

        wqb_ref[:, :] = wq_ref[:, :].astype(jnp.bfloat16)
        wob_ref[:, :] = wo_ref[:, :].astype(jnp.bfloat16)

        def compute_chunk(row):
            xb = x_ref[pl.ds(row, CH), :].astype(jnp.bfloat16)
            q_c = jnp.dot(xb, wqb_ref[:, :],
                          preferred_element_type=jnp.float32)
            q_c = (q_c * SCALE).astype(jnp.bfloat16)
            outs = []
            for h in range(n_local_heads):
                g = h // GQA_REP
                s = lax.dot_general(
                    q_c[:, h * Dh:(h + 1) * Dh], kb_ref[:, g * Dh:(g + 1) * Dh],
                    (((1,), (1,)), ((), ())),
                    preferred_element_type=jnp.float32)
                pj = jnp.exp(s).astype(jnp.bfloat16)
                l = jnp.sum(pj, axis=1, keepdims=True, dtype=jnp.float32)
                o = jnp.dot(pj, vb_ref[:, g * Dh:(g + 1) * Dh],
                            preferred_element_type=jnp.float32) / l
                outs.append(o)
            attn = jnp.concatenate(outs, axis=1).astype(jnp.bfloat16)
            return jnp.dot(attn, wob_ref[:, :],
                           preferred_element_type=jnp.float32)

        for delta in (2, 1, 3):
            o = mod4(p + delta)
            part = compute_chunk(CH * o)
            myslot, dstslot = delta - 1, 3 - delta
            sbuf_rs[myslot] = part.astype(jnp.bfloat16)
            rdma = pltpu.make_async_remote_copy(
                src_ref=sbuf_rs.at[myslot],
                dst_ref=stage_rs.at[dstslot],
                send_sem=rs_send.at[myslot], recv_sem=rs_recv.at[dstslot],
                device_id=(o,), device_id_type=pl.DeviceIdType.MESH,
            )
            rdma.start()

        own = compute_chunk(CH * p)

        for j in range(3):
            rr = pltpu.make_async_remote_copy(
                src_ref=stage_rs.at[j], dst_ref=stage_rs.at[j],
                send_sem=dummy_sem.at[j], recv_sem=rs_recv.at[j],
                device_id=(p,), device_id_type=pl.DeviceIdType.MESH,
            )
            rr.wait_recv()

        red = (own + stage_rs[0].astype(jnp.float32)
               + stage_rs[1].astype(jnp.float32)
               + stage_rs[2].astype(jnp.float32))
        out_ref[pl.ds(CH * p, CH), :] = red
        sbuf_ag[:, :] = red.astype(jnp.bfloat16)

        for delta in (1, 2, 3):
            rdma = pltpu.make_async_remote_copy(
                src_ref=sbuf_ag,
                dst_ref=stage_ag.at[3 - delta],
                send_sem=ag_send.at[delta - 1], recv_sem=ag_recv.at[3 - delta],
                device_id=(mod4(p + delta),),
                device_id_type=pl.DeviceIdType.MESH,
            )
            rdma.start()

        for j in range(3):
            ra = pltpu.make_async_remote_copy(
                src_ref=stage_ag.at[j], dst_ref=stage_ag.at[j],
                send_sem=dummy_sem.at[j], recv_sem=ag_recv.at[j],
                device_id=(p,), device_id_type=pl.DeviceIdType.MESH,
            )
            ra.wait_recv()
            src_owner = mod4(p + j + 1)
            out_ref[pl.ds(CH * src_owner, CH), :] = stage_ag[j].astype(jnp.float32)

        for j in range(3):
            ws = pltpu.make_async_remote_copy(
                src_ref=sbuf_rs.at[j], dst_ref=stage_rs.at[j],
                send_sem=rs_send.at[j], recv_sem=dummy_sem.at[j],
                device_id=(p,), device_id_type=pl.DeviceIdType.MESH,
            )
            ws.wait_send()
            wa = pltpu.make_async_remote_copy(
                src_ref=sbuf_ag, dst_ref=stage_ag.at[j],
                send_sem=ag_send.at[j], recv_sem=dummy_sem.at[j],
                device_id=(p,), device_id_type=pl.DeviceIdType.MESH,
            )
            wa.wait_send()

    out = pl.pallas_call(
        body,
        out_shape=jax.ShapeDtypeStruct((Sq, D), jnp.float32),
        in_specs=[pl.BlockSpec(memory_space=pltpu.VMEM)] * 5,
        out_specs=pl.BlockSpec(memory_space=pltpu.VMEM),
        scratch_shapes=[
            pltpu.VMEM((Wq.shape[0], Dq), jnp.bfloat16),
            pltpu.VMEM((Dq, D), jnp.bfloat16),
            pltpu.VMEM((3, CH, D), jnp.bfloat16),
            pltpu.VMEM((3, CH, D), jnp.bfloat16),
            pltpu.VMEM((CH, D), jnp.bfloat16),
            pltpu.VMEM((3, CH, D), jnp.bfloat16),
            pltpu.SemaphoreType.DMA((3,)),
            pltpu.SemaphoreType.DMA((3,)),
            pltpu.SemaphoreType.DMA((3,)),
            pltpu.SemaphoreType.DMA((3,)),
            pltpu.SemaphoreType.DMA((3,)),
        ],
        compiler_params=pltpu.CompilerParams(collective_id=0),
    )(xs, Wq, Wo, Ks, Vs)
    return out.reshape(B, Sq, D)
